# baseline (speedup 1.0000x reference)
import functools

import jax
import jax.numpy as jnp
from jax import lax
from jax.experimental import pallas as pl
from jax.experimental.pallas import tpu as pltpu
from jax.experimental.pallas import tpu_sc as plsc

V = 8192
D = 128
B = 64
L = 128
K = 32

NC = 2
NS = 16
NW = NC * NS
ROWS_PER_W = B // NW
LANES = 16
DC = D // LANES
KC = K // LANES
PB = 2


def _sc_body(emb_hbm, adjf_hbm, agg_hbm, nd_hbm, nb_hbm, rep_hbm,
             agg_v, nd_v, rate_v, ori_v, nb_v, nrow_v,
             erows_v, eidx_v, hval_v, acc_v,
             sem0, sem_e, sem_a):
    wid = lax.axis_index("s") * NC + lax.axis_index("c")
    row0 = wid * ROWS_PER_W

    pltpu.sync_copy(agg_hbm, agg_v)
    pltpu.sync_copy(nd_hbm.at[pl.ds(row0, ROWS_PER_W)], nd_v)
    pltpu.sync_copy(nb_hbm.at[pl.ds(row0, ROWS_PER_W)], nb_v)
    for bi in range(ROWS_PER_W):
        pltpu.async_copy(emb_hbm.at[nd_v.at[bi]], nrow_v.at[bi], sem0).wait()

    zero = jnp.zeros((LANES,), jnp.float32)
    for bi in range(ROWS_PER_W):
        for c in range(L // LANES):
            nchunk = nd_v[bi, pl.ds(c * LANES, LANES)]
            rate = plsc.load_gather(agg_v, [nchunk])
            pad = nchunk == 0
            rate_v[bi, pl.ds(c * LANES, LANES)] = jnp.where(pad, zero, rate)
            ori_v[bi, pl.ds(c * LANES, LANES)] = jnp.where(pad, zero, 1.0 - rate)

    for bi in range(ROWS_PER_W):
        for c in range(DC):
            acc_v[bi, pl.ds(c * LANES, LANES)] = zero

    def issue(bi, l, s):
        for j in range(PB):
            nspl = plsc.load_gather(nd_v.at[bi], [jnp.full((LANES,), j, jnp.int32) + l])
            for c in range(KC):
                nbc = nb_v[bi, pl.ds((l + j) * K + c * LANES, LANES)]
                eidx_v[s, pl.ds(j * K + c * LANES, LANES)] = nspl * V + nbc
        pltpu.async_copy(emb_hbm.at[nb_v.at[bi, pl.ds(l * K, PB * K)]],
                         erows_v.at[s], sem_e[s])
        pltpu.async_copy(adjf_hbm.at[eidx_v.at[s]], hval_v.at[s], sem_a[s])

    def wait(bi, l, s):
        pltpu.make_async_copy(emb_hbm.at[nb_v.at[bi, pl.ds(l * K, PB * K)]],
                              erows_v.at[s], sem_e[s]).wait()
        pltpu.make_async_copy(adjf_hbm.at[eidx_v.at[s]], hval_v.at[s],
                              sem_a[s]).wait()

    def compute(bi, l, s):
        for j in range(PB):
            m = [None] * DC
            for k in range(K):
                hk = plsc.load_gather(
                    hval_v.at[s], [jnp.full((LANES,), j * K + k, jnp.int32)])
                for c in range(DC):
                    p = hk * erows_v[s, j * K + k, pl.ds(c * LANES, LANES)]
                    m[c] = p if k == 0 else jnp.maximum(m[c], p)
            lsplat = jnp.full((LANES,), j, jnp.int32) + l
            ratesp = plsc.load_gather(rate_v.at[bi], [lsplat])
            orisp = plsc.load_gather(ori_v.at[bi], [lsplat])
            for c in range(DC):
                nrow = nrow_v[bi, l + j, pl.ds(c * LANES, LANES)]
                plsc.addupdate(acc_v.at[bi, pl.ds(c * LANES, LANES)],
                               ratesp * m[c] + orisp * nrow)

    for bi in range(ROWS_PER_W):
        issue(bi, 0, 0)
        issue(bi, PB, 1)

        def group(g, bi):
            for s in range(2):
                l = (g * 2 + s) * PB
                wait(bi, l, s)
                compute(bi, l, s)

                @pl.when(l + 2 * PB < L)
                def _():
                    issue(bi, l + 2 * PB, s)
            return bi

        lax.fori_loop(0, L // (2 * PB), group, bi)

    pltpu.sync_copy(acc_v, rep_hbm.at[pl.ds(row0, ROWS_PER_W)])


def _sc_aggregate(emb, adjf, agg, nodes, neighbors):
    mesh = plsc.VectorSubcoreMesh(core_axis_name="c", subcore_axis_name="s")
    f = functools.partial(
        pl.kernel,
        mesh=mesh,
        out_type=jax.ShapeDtypeStruct((B, D), jnp.float32),
        compiler_params=pltpu.CompilerParams(
            needs_layout_passes=False, use_tc_tiling_on_sc=False),
        scratch_types=[
            pltpu.VMEM((V,), jnp.float32),
            pltpu.VMEM((ROWS_PER_W, L), jnp.int32),
            pltpu.VMEM((ROWS_PER_W, L), jnp.float32),
            pltpu.VMEM((ROWS_PER_W, L), jnp.float32),
            pltpu.VMEM((ROWS_PER_W, L * K), jnp.int32),
            pltpu.VMEM((ROWS_PER_W, L, D), jnp.float32),
            pltpu.VMEM((2, PB * K, D), jnp.float32),
            pltpu.VMEM((2, PB * K), jnp.int32),
            pltpu.VMEM((2, PB * K), jnp.float32),
            pltpu.VMEM((ROWS_PER_W, D), jnp.float32),
            pltpu.SemaphoreType.DMA,
            [pltpu.SemaphoreType.DMA] * 2,
            [pltpu.SemaphoreType.DMA] * 2,
        ],
    )(_sc_body)
    return f(emb, adjf, agg, nodes, neighbors)


def _tc_head_body(rep_ref, w_ref, b_ref, o_ref):
    o_ref[...] = (
        jnp.dot(rep_ref[...], w_ref[...], preferred_element_type=jnp.float32)
        + b_ref[...]
    )


def _tc_head(rep, W_last, b_last):
    return pl.pallas_call(
        _tc_head_body,
        out_shape=jax.ShapeDtypeStruct((B, 2), jnp.float32),
    )(rep, W_last, b_last.reshape(1, 2))


def kernel(nodes, neighbors, adjacency_matrix, embedding_table,
           aggregate_rate, W_last, b_last):
    nodes = nodes.astype(jnp.int32)
    neighbors = neighbors.astype(jnp.int32)
    rep = _sc_aggregate(embedding_table, adjacency_matrix.reshape(V * V),
                        aggregate_rate, nodes, neighbors.reshape(B, L * K))
    return _tc_head(rep, W_last, b_last)


# speedup vs baseline: 1.7883x; 1.7883x over previous
import functools

import jax
import jax.numpy as jnp
from jax import lax
from jax.experimental import pallas as pl
from jax.experimental.pallas import tpu as pltpu
from jax.experimental.pallas import tpu_sc as plsc

V = 8192
D = 128
B = 64
L = 128
K = 32

NC = 2
NS = 16
NW = NC * NS
ROWS_PER_W = B // NW
LANES = 16
DC = D // LANES
KC = K // LANES


def _sc_body(emb_hbm, adj_hbm, agg_hbm, nd_hbm, nb_hbm, rep_hbm,
             agg_v, nd_v, rate_v, ori_v, nb_v, nrow_v,
             erows_v, arows_v, h_v, acc_v,
             sem0, sem_e, sem_a):
    wid = lax.axis_index("s") * NC + lax.axis_index("c")
    row0 = wid * ROWS_PER_W

    cp_agg = pltpu.async_copy(agg_hbm, agg_v, sem0[0])
    cp_nd = pltpu.async_copy(nd_hbm.at[pl.ds(row0, ROWS_PER_W)], nd_v, sem0[1])
    cp_nb = pltpu.async_copy(nb_hbm.at[pl.ds(row0, ROWS_PER_W)], nb_v, sem0[2])
    cp_nd.wait()
    nrow_cps = [pltpu.async_copy(emb_hbm.at[nd_v.at[bi]], nrow_v.at[bi],
                                 sem0[3 + bi])
                for bi in range(ROWS_PER_W)]
    cp_agg.wait()

    zero = jnp.zeros((LANES,), jnp.float32)
    for bi in range(ROWS_PER_W):
        for c in range(L // LANES):
            nchunk = nd_v[bi, pl.ds(c * LANES, LANES)]
            rate = plsc.load_gather(agg_v, [nchunk])
            pad = nchunk == 0
            rate_v[bi, pl.ds(c * LANES, LANES)] = jnp.where(pad, zero, rate)
            ori_v[bi, pl.ds(c * LANES, LANES)] = jnp.where(pad, zero, 1.0 - rate)

    for bi in range(ROWS_PER_W):
        for c in range(DC):
            acc_v[bi, pl.ds(c * LANES, LANES)] = zero

    cp_nb.wait()
    for cp in nrow_cps:
        cp.wait()

    zeros16i = jnp.zeros((LANES,), jnp.int32)

    def issue(bi, l, s):
        pltpu.async_copy(emb_hbm.at[nb_v.at[bi, l]], erows_v.at[s], sem_e[s])
        pltpu.async_copy(adj_hbm.at[nd_v.at[bi, pl.ds(l, 1)]], arows_v.at[s],
                         sem_a[s])

    def wait(bi, l, s):
        pltpu.make_async_copy(emb_hbm.at[nb_v.at[bi, l]], erows_v.at[s],
                              sem_e[s]).wait()
        pltpu.make_async_copy(adj_hbm.at[nd_v.at[bi, pl.ds(l, 1)]],
                              arows_v.at[s], sem_a[s]).wait()

    def compute(bi, l, s):
        for c in range(KC):
            nbc = nb_v[bi, l, pl.ds(c * LANES, LANES)]
            h_v[pl.ds(c * LANES, LANES)] = plsc.load_gather(
                arows_v.at[s], [zeros16i, nbc])
        m = [None] * DC
        for k in range(K):
            hk = plsc.load_gather(h_v, [jnp.full((LANES,), k, jnp.int32)])
            for c in range(DC):
                p = hk * erows_v[s, k, pl.ds(c * LANES, LANES)]
                m[c] = p if k == 0 else jnp.maximum(m[c], p)
        lsplat = jnp.full((LANES,), 0, jnp.int32) + l
        ratesp = plsc.load_gather(rate_v.at[bi], [lsplat])
        orisp = plsc.load_gather(ori_v.at[bi], [lsplat])
        for c in range(DC):
            nrow = nrow_v[bi, l, pl.ds(c * LANES, LANES)]
            plsc.addupdate(acc_v.at[bi, pl.ds(c * LANES, LANES)],
                           ratesp * m[c] + orisp * nrow)

    for bi in range(ROWS_PER_W):
        issue(bi, 0, 0)
        issue(bi, 1, 1)

        def group(g, bi):
            for s in range(2):
                l = g * 2 + s
                wait(bi, l, s)
                compute(bi, l, s)

                @pl.when(l + 2 < L)
                def _():
                    issue(bi, l + 2, s)
            return bi

        lax.fori_loop(0, L // 2, group, bi)

    pltpu.sync_copy(acc_v, rep_hbm.at[pl.ds(row0, ROWS_PER_W)])


def _sc_aggregate(emb, adj, agg, nodes, neighbors):
    mesh = plsc.VectorSubcoreMesh(core_axis_name="c", subcore_axis_name="s")
    f = functools.partial(
        pl.kernel,
        mesh=mesh,
        out_type=jax.ShapeDtypeStruct((B, D), jnp.float32),
        compiler_params=pltpu.CompilerParams(needs_layout_passes=False),
        scratch_types=[
            pltpu.VMEM((V,), jnp.float32),
            pltpu.VMEM((ROWS_PER_W, L), jnp.int32),
            pltpu.VMEM((ROWS_PER_W, L), jnp.float32),
            pltpu.VMEM((ROWS_PER_W, L), jnp.float32),
            pltpu.VMEM((ROWS_PER_W, L, K), jnp.int32),
            pltpu.VMEM((ROWS_PER_W, L, D), jnp.float32),
            pltpu.VMEM((2, K, D), jnp.float32),
            pltpu.VMEM((2, 1, V), jnp.float32),
            pltpu.VMEM((K,), jnp.float32),
            pltpu.VMEM((ROWS_PER_W, D), jnp.float32),
            [pltpu.SemaphoreType.DMA] * 5,
            [pltpu.SemaphoreType.DMA] * 2,
            [pltpu.SemaphoreType.DMA] * 2,
        ],
    )(_sc_body)
    return f(emb, adj, agg, nodes, neighbors)


def _tc_head_body(rep_ref, w_ref, b_ref, o_ref):
    o_ref[...] = (
        jnp.dot(rep_ref[...], w_ref[...], preferred_element_type=jnp.float32)
        + b_ref[...]
    )


def _tc_head(rep, W_last, b_last):
    return pl.pallas_call(
        _tc_head_body,
        out_shape=jax.ShapeDtypeStruct((B, 2), jnp.float32),
    )(rep, W_last, b_last.reshape(1, 2))


def kernel(nodes, neighbors, adjacency_matrix, embedding_table,
           aggregate_rate, W_last, b_last):
    nodes = nodes.astype(jnp.int32)
    neighbors = neighbors.astype(jnp.int32)
    rep = _sc_aggregate(embedding_table, adjacency_matrix, aggregate_rate,
                        nodes, neighbors)
    return _tc_head(rep, W_last, b_last)
